# baseline (speedup 1.0000x reference)
import functools

import jax
import jax.numpy as jnp
from jax import lax
from jax.experimental import pallas as pl
from jax.experimental.pallas import tpu as pltpu
from jax.experimental.pallas import tpu_sc as plsc

N = 10000
E = 160000
D = 256
DH = D // 2
NC = 2
NS = 16
EPT = E // NS
K = 80
NB = EPT // K
NSETS = 4
STRIPE = 624
TAIL = N - NS * STRIPE

BN = 1000
NBLK = N // BN


def _edge_phase(xh, sd, ea3):
    mesh = plsc.VectorSubcoreMesh(core_axis_name="c", subcore_axis_name="s")

    @functools.partial(
        pl.kernel,
        mesh=mesh,
        out_type=jax.ShapeDtypeStruct((NC, N, DH), jnp.float32),
        scratch_types=[
            [pltpu.VMEM((2, K), jnp.int32)] * NSETS,
            [pltpu.VMEM((K, DH), jnp.float32)] * NSETS,
            pltpu.VMEM_SHARED((N, DH), jnp.float32),
            [pltpu.SemaphoreType.DMA] * NSETS,
            [pltpu.SemaphoreType.DMA] * NSETS,
            [pltpu.SemaphoreType.DMA] * NSETS,
            [pltpu.SemaphoreType.DMA] * NSETS,
        ],
    )
    def sc_kernel(xh_hbm, sd_hbm, ea_hbm, hh_hbm,
                  sd_v, buf, agg_sh, isem, esem, gsem, ssem):
        c = lax.axis_index("c")
        s = lax.axis_index("s")

        r0 = s * STRIPE
        pltpu.sync_copy(xh_hbm.at[c, pl.ds(r0, STRIPE)],
                        agg_sh.at[pl.ds(r0, STRIPE)])

        @pl.when(s == NS - 1)
        def _():
            pltpu.sync_copy(xh_hbm.at[c, pl.ds(NS * STRIPE, TAIL)],
                            agg_sh.at[pl.ds(NS * STRIPE, TAIL)])

        plsc.subcore_barrier()

        def start_idx(b, p):
            pltpu.async_copy(sd_hbm.at[s, b], sd_v[p], isem[p])

        def wait_idx(b, p):
            pltpu.make_async_copy(sd_hbm.at[s, b], sd_v[p], isem[p]).wait()

        def start_ea(b, p):
            pltpu.async_copy(
                ea_hbm.at[pl.ds(s * EPT + b * K, K), pl.ds(c * DH, DH)], buf[p], esem[p])

        def wait_ea(b, p):
            pltpu.make_async_copy(
                ea_hbm.at[pl.ds(s * EPT + b * K, K), pl.ds(c * DH, DH)], buf[p],
                esem[p]).wait()

        def start_gather_add(p):
            pltpu.async_copy(xh_hbm.at[c].at[sd_v[p].at[0]], buf[p],
                             gsem[p], add=True)

        def wait_gather(p):
            pltpu.make_async_copy(xh_hbm.at[c].at[sd_v[p].at[0]], buf[p],
                                  gsem[p]).wait()

        def start_scat(p):
            pltpu.async_copy(buf[p], agg_sh.at[sd_v[p].at[1]], ssem[p],
                             add=True)

        def wait_scat(p):
            pltpu.make_async_copy(buf[p], agg_sh.at[sd_v[p].at[1]],
                                  ssem[p]).wait()

        def relu(p):
            @plsc.parallel_loop(0, K, step=2, unroll=4)
            def _(r):
                for rr in range(2):
                    for j in range(DH // 16):
                        sl = pl.ds(j * 16, 16)
                        buf[p][r + rr, sl] = jnp.maximum(buf[p][r + rr, sl],
                                                         0.0)

        start_idx(0, 0)
        start_ea(0, 0)
        start_idx(1, 1)
        start_ea(1, 1)
        wait_idx(0, 0)
        wait_ea(0, 0)
        start_gather_add(0)

        def quad_body(b4, carry):
            b0 = b4 * NSETS
            for i in range(NSETS):
                b = b0 + i
                p = i
                q1 = (i + 1) % NSETS
                q2 = (i + 2) % NSETS

                def far(bq2):
                    @pl.when(bq2 >= NSETS)
                    def _():
                        wait_scat(q2)
                    start_idx(bq2, q2)
                    start_ea(bq2, q2)

                if i < NSETS - 2:
                    far(b + 2)
                else:
                    @pl.when(b + 2 < NB)
                    def _(bq2=b + 2):
                        far(bq2)

                def near(bq1):
                    wait_idx(bq1, q1)
                    wait_ea(bq1, q1)
                    start_gather_add(q1)

                if i < NSETS - 1:
                    near(b + 1)
                else:
                    @pl.when(b + 1 < NB)
                    def _(bq1=b + 1):
                        near(bq1)

                wait_gather(p)
                relu(p)
                start_scat(p)
            return carry

        lax.fori_loop(0, NB // NSETS, quad_body, 0, unroll=False)

        for tb in range(NB - NB % NSETS, NB):
            tp = tb % NSETS
            wait_gather(tp)
            relu(tp)
            start_scat(tp)

        for i in range(NSETS):
            wait_scat((NB - NSETS + i) % NSETS)

        plsc.subcore_barrier()

        pltpu.sync_copy(agg_sh.at[pl.ds(r0, STRIPE)],
                        hh_hbm.at[c, pl.ds(r0, STRIPE)])

        @pl.when(s == NS - 1)
        def _():
            pltpu.sync_copy(agg_sh.at[pl.ds(NS * STRIPE, TAIL)],
                            hh_hbm.at[c, pl.ds(NS * STRIPE, TAIL)])

    return sc_kernel(xh, sd, ea3)


def _split_kernel(x_ref, xh_ref):
    xh_ref[0] = x_ref[:, :DH]
    xh_ref[1] = x_ref[:, DH:]


def _mlp_kernel(hh_ref, w1_ref, b1_ref, w2_ref, b2_ref, h2_ref, sums_ref):
    h = jnp.concatenate([hh_ref[0], hh_ref[1]], axis=1)
    t = lax.dot_general(h, w1_ref[...], (((1,), (1,)), ((), ())),
                        preferred_element_type=jnp.float32)
    t = jnp.maximum(t + b1_ref[...], 0.0)
    h2 = lax.dot_general(t, w2_ref[...], (((1,), (1,)), ((), ())),
                         preferred_element_type=jnp.float32)
    h2 = h2 + b2_ref[...]
    h2_ref[...] = h2
    s1 = jnp.sum(h2)
    s2 = jnp.sum(h2 * h2)
    col = lax.broadcasted_iota(jnp.int32, (1, 1, 128), 2)
    sums_ref[...] = jnp.where(col == 0, s1, jnp.where(col == 1, s2, 0.0))


def _epilogue_kernel(h2_ref, x_ref, sums_ref, lnw_ref, lnb_ref, out_ref):
    s1 = jnp.sum(sums_ref[:, :, 0:1])
    s2 = jnp.sum(sums_ref[:, :, 1:2])
    cnt = jnp.float32(N * D)
    mu = s1 / cnt
    var = s2 / cnt - mu * mu
    denom = jnp.sqrt(var) + jnp.float32(1e-5)
    h2 = h2_ref[...]
    out = (h2 - mu) / denom * lnw_ref[...] + lnb_ref[...]
    out_ref[...] = jnp.maximum(out, 0.0) + x_ref[...]


def kernel(x, edge_index, edge_attr, W1, b1, W2, b2, ln_w, ln_b):
    sd = edge_index.astype(jnp.int32).reshape(2, NS, NB, K).transpose(
        1, 2, 0, 3)
    xh = pl.pallas_call(
        _split_kernel,
        grid=(NBLK,),
        in_specs=[pl.BlockSpec((BN, D), lambda i: (i, 0))],
        out_specs=pl.BlockSpec((NC, BN, DH), lambda i: (0, i, 0)),
        out_shape=jax.ShapeDtypeStruct((NC, N, DH), jnp.float32),
    )(x)
    hh = _edge_phase(xh, sd, edge_attr)

    h2, sums = pl.pallas_call(
        _mlp_kernel,
        grid=(NBLK,),
        in_specs=[
            pl.BlockSpec((NC, BN, DH), lambda i: (0, i, 0)),
            pl.BlockSpec((D, D), lambda i: (0, 0)),
            pl.BlockSpec((1, D), lambda i: (0, 0)),
            pl.BlockSpec((D, D), lambda i: (0, 0)),
            pl.BlockSpec((1, D), lambda i: (0, 0)),
        ],
        out_specs=[
            pl.BlockSpec((BN, D), lambda i: (i, 0)),
            pl.BlockSpec((1, 1, 128), lambda i: (i, 0, 0)),
        ],
        out_shape=[
            jax.ShapeDtypeStruct((N, D), jnp.float32),
            jax.ShapeDtypeStruct((NBLK, 1, 128), jnp.float32),
        ],
    )(hh, W1, b1.reshape(1, D), W2, b2.reshape(1, D))

    out = pl.pallas_call(
        _epilogue_kernel,
        grid=(NBLK,),
        in_specs=[
            pl.BlockSpec((BN, D), lambda i: (i, 0)),
            pl.BlockSpec((BN, D), lambda i: (i, 0)),
            pl.BlockSpec((NBLK, 1, 128), lambda i: (0, 0, 0)),
            pl.BlockSpec((1, D), lambda i: (0, 0)),
            pl.BlockSpec((1, D), lambda i: (0, 0)),
        ],
        out_specs=pl.BlockSpec((BN, D), lambda i: (i, 0)),
        out_shape=jax.ShapeDtypeStruct((N, D), jnp.float32),
    )(h2, x, sums, ln_w.reshape(1, D), ln_b.reshape(1, D))

    return out


# speedup vs baseline: 1.0256x; 1.0256x over previous
import functools

import jax
import jax.numpy as jnp
from jax import lax
from jax.experimental import pallas as pl
from jax.experimental.pallas import tpu as pltpu
from jax.experimental.pallas import tpu_sc as plsc

N = 10000
E = 160000
D = 256
DH = D // 2
NC = 2
NS = 16
EPT = E // NS
K = 80
NB = EPT // K
NSETS = 4
STRIPE = 624
TAIL = N - NS * STRIPE

BN = 1000
NBLK = N // BN


def _edge_phase(xh, sd, ea3):
    mesh = plsc.VectorSubcoreMesh(core_axis_name="c", subcore_axis_name="s")

    @functools.partial(
        pl.kernel,
        mesh=mesh,
        out_type=jax.ShapeDtypeStruct((NC, N, DH), jnp.float32),
        scratch_types=[
            [pltpu.VMEM((2, K), jnp.int32)] * NSETS,
            [pltpu.VMEM((K, DH), jnp.float32)] * NSETS,
            pltpu.VMEM_SHARED((N, DH), jnp.float32),
            [pltpu.SemaphoreType.DMA] * NSETS,
            [pltpu.SemaphoreType.DMA] * NSETS,
            [pltpu.SemaphoreType.DMA] * NSETS,
            [pltpu.SemaphoreType.DMA] * NSETS,
        ],
    )
    def sc_kernel(xh_hbm, sd_hbm, ea_hbm, hh_hbm,
                  sd_v, buf, agg_sh, isem, esem, gsem, ssem):
        c = lax.axis_index("c")
        s = lax.axis_index("s")

        r0 = s * STRIPE
        pltpu.sync_copy(xh_hbm.at[c, pl.ds(r0, STRIPE)],
                        agg_sh.at[pl.ds(r0, STRIPE)])

        @pl.when(s == NS - 1)
        def _():
            pltpu.sync_copy(xh_hbm.at[c, pl.ds(NS * STRIPE, TAIL)],
                            agg_sh.at[pl.ds(NS * STRIPE, TAIL)])

        plsc.subcore_barrier()

        def start_idx(b, p):
            pltpu.async_copy(sd_hbm.at[s, b], sd_v[p], isem[p])

        def wait_idx(b, p):
            pltpu.make_async_copy(sd_hbm.at[s, b], sd_v[p], isem[p]).wait()

        def start_ea(b, p):
            pltpu.async_copy(
                ea_hbm.at[pl.ds(s * EPT + b * K, K), pl.ds(c * DH, DH)], buf[p], esem[p])

        def wait_ea(b, p):
            pltpu.make_async_copy(
                ea_hbm.at[pl.ds(s * EPT + b * K, K), pl.ds(c * DH, DH)], buf[p],
                esem[p]).wait()

        def start_gather_add(p):
            pltpu.async_copy(xh_hbm.at[c].at[sd_v[p].at[0]], buf[p],
                             gsem[p], add=True)

        def wait_gather(p):
            pltpu.make_async_copy(xh_hbm.at[c].at[sd_v[p].at[0]], buf[p],
                                  gsem[p]).wait()

        def start_scat(p):
            pltpu.async_copy(buf[p], agg_sh.at[sd_v[p].at[1]], ssem[p],
                             add=True)

        def wait_scat(p):
            pltpu.make_async_copy(buf[p], agg_sh.at[sd_v[p].at[1]],
                                  ssem[p]).wait()

        def relu(p):
            @plsc.parallel_loop(0, K, step=2, unroll=4)
            def _(r):
                for rr in range(2):
                    for j in range(DH // 16):
                        sl = pl.ds(j * 16, 16)
                        buf[p][r + rr, sl] = jnp.maximum(buf[p][r + rr, sl],
                                                         0.0)

        start_idx(0, 0)
        start_ea(0, 0)
        start_idx(1, 1)
        start_ea(1, 1)
        wait_idx(0, 0)
        wait_ea(0, 0)
        start_gather_add(0)

        def quad_body(b4, carry):
            b0 = b4 * NSETS
            for i in range(NSETS):
                b = b0 + i
                p = i
                q1 = (i + 1) % NSETS
                q2 = (i + 2) % NSETS

                def far(bq2):
                    @pl.when(bq2 >= NSETS)
                    def _():
                        wait_scat(q2)
                    start_idx(bq2, q2)
                    start_ea(bq2, q2)

                if i < NSETS - 2:
                    far(b + 2)
                else:
                    @pl.when(b + 2 < NB)
                    def _(bq2=b + 2):
                        far(bq2)

                def near(bq1):
                    wait_idx(bq1, q1)
                    wait_ea(bq1, q1)
                    start_gather_add(q1)

                if i < NSETS - 1:
                    near(b + 1)
                else:
                    @pl.when(b + 1 < NB)
                    def _(bq1=b + 1):
                        near(bq1)

                wait_gather(p)
                relu(p)
                start_scat(p)
            return carry

        lax.fori_loop(0, NB // NSETS, quad_body, 0, unroll=False)

        for tb in range(NB - NB % NSETS, NB):
            tp = tb % NSETS
            wait_gather(tp)
            relu(tp)
            start_scat(tp)

        for i in range(NSETS):
            wait_scat((NB - NSETS + i) % NSETS)

        plsc.subcore_barrier()

        pltpu.sync_copy(agg_sh.at[pl.ds(r0, STRIPE)],
                        hh_hbm.at[c, pl.ds(r0, STRIPE)])

        @pl.when(s == NS - 1)
        def _():
            pltpu.sync_copy(agg_sh.at[pl.ds(NS * STRIPE, TAIL)],
                            hh_hbm.at[c, pl.ds(NS * STRIPE, TAIL)])

    return sc_kernel(xh, sd, ea3)


def _split_kernel(x_ref, xh_ref):
    xh_ref[0] = x_ref[:, :DH]
    xh_ref[1] = x_ref[:, DH:]


def _fused_mlp_ln_kernel(hh_ref, x_ref, w1_ref, b1_ref, w2_ref, b2_ref,
                         lnw_ref, lnb_ref, out_ref, h2_scr, sums_smem):
    ph = pl.program_id(0)
    i = pl.program_id(1)

    @pl.when(ph == 0)
    def _():
        h = jnp.concatenate([hh_ref[0], hh_ref[1]], axis=1)
        t = lax.dot_general(h, w1_ref[...], (((1,), (1,)), ((), ())),
                            preferred_element_type=jnp.float32)
        t = jnp.maximum(t + b1_ref[...], 0.0)
        h2 = lax.dot_general(t, w2_ref[...], (((1,), (1,)), ((), ())),
                             preferred_element_type=jnp.float32)
        h2 = h2 + b2_ref[...]
        h2_scr[pl.ds(i * BN, BN), :] = h2

        @pl.when(i == 0)
        def _():
            sums_smem[0] = 0.0
            sums_smem[1] = 0.0

        sums_smem[0] += jnp.sum(h2)
        sums_smem[1] += jnp.sum(h2 * h2)

    @pl.when(ph == 1)
    def _():
        cnt = jnp.float32(N * D)
        mu = sums_smem[0] / cnt
        var = sums_smem[1] / cnt - mu * mu
        denom = jnp.sqrt(var) + jnp.float32(1e-5)
        h2 = h2_scr[pl.ds(i * BN, BN), :]
        out = (h2 - mu) / denom * lnw_ref[...] + lnb_ref[...]
        out_ref[...] = jnp.maximum(out, 0.0) + x_ref[...]


def kernel(x, edge_index, edge_attr, W1, b1, W2, b2, ln_w, ln_b):
    sd = edge_index.astype(jnp.int32).reshape(2, NS, NB, K).transpose(
        1, 2, 0, 3)
    xh = pl.pallas_call(
        _split_kernel,
        grid=(NBLK,),
        in_specs=[pl.BlockSpec((BN, D), lambda i: (i, 0))],
        out_specs=pl.BlockSpec((NC, BN, DH), lambda i: (0, i, 0)),
        out_shape=jax.ShapeDtypeStruct((NC, N, DH), jnp.float32),
    )(x)
    hh = _edge_phase(xh, sd, edge_attr)

    out = pl.pallas_call(
        _fused_mlp_ln_kernel,
        grid=(2, NBLK),
        in_specs=[
            pl.BlockSpec((NC, BN, DH),
                         lambda ph, i: (0, jnp.where(ph == 0, i, NBLK - 1), 0)),
            pl.BlockSpec((BN, D), lambda ph, i: (jnp.where(ph == 0, 0, i), 0)),
            pl.BlockSpec((D, D), lambda ph, i: (0, 0)),
            pl.BlockSpec((1, D), lambda ph, i: (0, 0)),
            pl.BlockSpec((D, D), lambda ph, i: (0, 0)),
            pl.BlockSpec((1, D), lambda ph, i: (0, 0)),
            pl.BlockSpec((1, D), lambda ph, i: (0, 0)),
            pl.BlockSpec((1, D), lambda ph, i: (0, 0)),
        ],
        out_specs=pl.BlockSpec((BN, D), lambda ph, i: (jnp.where(ph == 0, 0, i), 0)),
        out_shape=jax.ShapeDtypeStruct((N, D), jnp.float32),
        scratch_shapes=[
            pltpu.VMEM((N, D), jnp.float32),
            pltpu.SMEM((2,), jnp.float32),
        ],
    )(hh, x, W1, b1.reshape(1, D), W2, b2.reshape(1, D),
      ln_w.reshape(1, D), ln_b.reshape(1, D))

    return out


# speedup vs baseline: 1.0262x; 1.0006x over previous
import functools

import jax
import jax.numpy as jnp
from jax import lax
from jax.experimental import pallas as pl
from jax.experimental.pallas import tpu as pltpu
from jax.experimental.pallas import tpu_sc as plsc

N = 10000
E = 160000
D = 256
DH = D // 2
NC = 2
NS = 16
EPT = E // NS
K = 80
NB = EPT // K
NSETS = 4
STRIPE = 624
TAIL = N - NS * STRIPE

BN = 1000
NBLK = N // BN


def _edge_phase(xh, sd, ea3):
    mesh = plsc.VectorSubcoreMesh(core_axis_name="c", subcore_axis_name="s")

    @functools.partial(
        pl.kernel,
        mesh=mesh,
        out_type=jax.ShapeDtypeStruct((NC, N, DH), jnp.float32),
        scratch_types=[
            [pltpu.VMEM((2, K), jnp.int32)] * NSETS,
            [pltpu.VMEM((K, DH), jnp.float32)] * NSETS,
            pltpu.VMEM_SHARED((N, DH), jnp.float32),
            [pltpu.SemaphoreType.DMA] * NSETS,
            [pltpu.SemaphoreType.DMA] * NSETS,
            [pltpu.SemaphoreType.DMA] * NSETS,
            [pltpu.SemaphoreType.DMA] * NSETS,
        ],
    )
    def sc_kernel(xh_hbm, sd_hbm, ea_hbm, hh_hbm,
                  sd_v, buf, agg_sh, isem, esem, gsem, ssem):
        c = lax.axis_index("c")
        s = lax.axis_index("s")

        r0 = s * STRIPE
        pltpu.sync_copy(xh_hbm.at[c, pl.ds(r0, STRIPE)],
                        agg_sh.at[pl.ds(r0, STRIPE)])

        @pl.when(s == NS - 1)
        def _():
            pltpu.sync_copy(xh_hbm.at[c, pl.ds(NS * STRIPE, TAIL)],
                            agg_sh.at[pl.ds(NS * STRIPE, TAIL)])

        plsc.subcore_barrier()

        def start_idx(b, p):
            pltpu.async_copy(sd_hbm.at[s, b], sd_v[p], isem[p])

        def wait_idx(b, p):
            pltpu.make_async_copy(sd_hbm.at[s, b], sd_v[p], isem[p]).wait()

        def start_ea(b, p):
            pltpu.async_copy(
                ea_hbm.at[pl.ds(s * EPT + b * K, K), pl.ds(c * DH, DH)], buf[p], esem[p])

        def wait_ea(b, p):
            pltpu.make_async_copy(
                ea_hbm.at[pl.ds(s * EPT + b * K, K), pl.ds(c * DH, DH)], buf[p],
                esem[p]).wait()

        def start_gather_add(p):
            pltpu.async_copy(xh_hbm.at[c].at[sd_v[p].at[0]], buf[p],
                             gsem[p], add=True)

        def wait_gather(p):
            pltpu.make_async_copy(xh_hbm.at[c].at[sd_v[p].at[0]], buf[p],
                                  gsem[p]).wait()

        def start_scat(p):
            pltpu.async_copy(buf[p], agg_sh.at[sd_v[p].at[1]], ssem[p],
                             add=True)

        def wait_scat(p):
            pltpu.make_async_copy(buf[p], agg_sh.at[sd_v[p].at[1]],
                                  ssem[p]).wait()

        def relu(p):
            @plsc.parallel_loop(0, K, step=2, unroll=4)
            def _(r):
                for rr in range(2):
                    for j in range(DH // 16):
                        sl = pl.ds(j * 16, 16)
                        buf[p][r + rr, sl] = jnp.maximum(buf[p][r + rr, sl],
                                                         0.0)

        start_idx(0, 0)
        start_ea(0, 0)
        start_idx(1, 1)
        start_ea(1, 1)
        wait_idx(0, 0)
        wait_ea(0, 0)
        start_gather_add(0)

        def quad_body(b4, carry):
            b0 = b4 * NSETS
            for i in range(NSETS):
                b = b0 + i
                p = i
                q1 = (i + 1) % NSETS
                q2 = (i + 2) % NSETS

                def far(bq2):
                    @pl.when(bq2 >= NSETS)
                    def _():
                        wait_scat(q2)
                    start_idx(bq2, q2)
                    start_ea(bq2, q2)

                if i < NSETS - 2:
                    far(b + 2)
                else:
                    @pl.when(b + 2 < NB)
                    def _(bq2=b + 2):
                        far(bq2)

                def near(bq1):
                    wait_idx(bq1, q1)
                    wait_ea(bq1, q1)
                    start_gather_add(q1)

                if i < NSETS - 1:
                    near(b + 1)
                else:
                    @pl.when(b + 1 < NB)
                    def _(bq1=b + 1):
                        near(bq1)

                wait_gather(p)
                relu(p)
                start_scat(p)
            return carry

        lax.fori_loop(0, NB // NSETS, quad_body, 0, unroll=False)

        for tb in range(NB - NB % NSETS, NB):
            tp = tb % NSETS
            wait_gather(tp)
            relu(tp)
            start_scat(tp)

        for i in range(NSETS):
            wait_scat((NB - NSETS + i) % NSETS)

        plsc.subcore_barrier()

        pltpu.sync_copy(agg_sh.at[pl.ds(r0, STRIPE)],
                        hh_hbm.at[c, pl.ds(r0, STRIPE)])

        @pl.when(s == NS - 1)
        def _():
            pltpu.sync_copy(agg_sh.at[pl.ds(NS * STRIPE, TAIL)],
                            hh_hbm.at[c, pl.ds(NS * STRIPE, TAIL)])

    return sc_kernel(xh, sd, ea3)


def _split_kernel(x_ref, xh_ref):
    xh_ref[0] = x_ref[:, :DH]
    xh_ref[1] = x_ref[:, DH:]


def _fused_mlp_ln_kernel(hh_ref, x_ref, w1_ref, b1_ref, w2_ref, b2_ref,
                         lnw_ref, lnb_ref, out_ref, h2_scr, sums_smem):
    ph = pl.program_id(0)
    i = pl.program_id(1)

    @pl.when(ph == 0)
    def _():
        h = jnp.concatenate([hh_ref[0], hh_ref[1]], axis=1).astype(jnp.bfloat16)
        t = lax.dot_general(h, w1_ref[...].astype(jnp.bfloat16),
                            (((1,), (1,)), ((), ())),
                            preferred_element_type=jnp.float32)
        t = jnp.maximum(t + b1_ref[...], 0.0).astype(jnp.bfloat16)
        h2 = lax.dot_general(t, w2_ref[...].astype(jnp.bfloat16),
                             (((1,), (1,)), ((), ())),
                             preferred_element_type=jnp.float32)
        h2 = h2 + b2_ref[...]
        h2_scr[pl.ds(i * BN, BN), :] = h2

        @pl.when(i == 0)
        def _():
            sums_smem[0] = 0.0
            sums_smem[1] = 0.0

        sums_smem[0] += jnp.sum(h2)
        sums_smem[1] += jnp.sum(h2 * h2)

    @pl.when(ph == 1)
    def _():
        cnt = jnp.float32(N * D)
        mu = sums_smem[0] / cnt
        var = sums_smem[1] / cnt - mu * mu
        denom = jnp.sqrt(var) + jnp.float32(1e-5)
        h2 = h2_scr[pl.ds(i * BN, BN), :]
        out = (h2 - mu) / denom * lnw_ref[...] + lnb_ref[...]
        out_ref[...] = jnp.maximum(out, 0.0) + x_ref[...]


def kernel(x, edge_index, edge_attr, W1, b1, W2, b2, ln_w, ln_b):
    sd = edge_index.astype(jnp.int32).reshape(2, NS, NB, K).transpose(
        1, 2, 0, 3)
    xh = pl.pallas_call(
        _split_kernel,
        grid=(NBLK,),
        in_specs=[pl.BlockSpec((BN, D), lambda i: (i, 0))],
        out_specs=pl.BlockSpec((NC, BN, DH), lambda i: (0, i, 0)),
        out_shape=jax.ShapeDtypeStruct((NC, N, DH), jnp.float32),
    )(x)
    hh = _edge_phase(xh, sd, edge_attr)

    out = pl.pallas_call(
        _fused_mlp_ln_kernel,
        grid=(2, NBLK),
        in_specs=[
            pl.BlockSpec((NC, BN, DH),
                         lambda ph, i: (0, jnp.where(ph == 0, i, NBLK - 1), 0)),
            pl.BlockSpec((BN, D), lambda ph, i: (jnp.where(ph == 0, 0, i), 0)),
            pl.BlockSpec((D, D), lambda ph, i: (0, 0)),
            pl.BlockSpec((1, D), lambda ph, i: (0, 0)),
            pl.BlockSpec((D, D), lambda ph, i: (0, 0)),
            pl.BlockSpec((1, D), lambda ph, i: (0, 0)),
            pl.BlockSpec((1, D), lambda ph, i: (0, 0)),
            pl.BlockSpec((1, D), lambda ph, i: (0, 0)),
        ],
        out_specs=pl.BlockSpec((BN, D), lambda ph, i: (jnp.where(ph == 0, 0, i), 0)),
        out_shape=jax.ShapeDtypeStruct((N, D), jnp.float32),
        scratch_shapes=[
            pltpu.VMEM((N, D), jnp.float32),
            pltpu.SMEM((2,), jnp.float32),
        ],
    )(hh, x, W1, b1.reshape(1, D), W2, b2.reshape(1, D),
      ln_w.reshape(1, D), ln_b.reshape(1, D))

    return out
